# baseline (speedup 1.0000x reference)
import jax
import jax.numpy as jnp
from jax.experimental import pallas as pl
from jax.experimental.pallas import tpu as pltpu

_B = 4
_S = 8193
_D = 2048
_ROWS = _B * _S
_BLOCK = 256
_NB = (_ROWS + _BLOCK - 1) // _BLOCK
_LAST = _ROWS - (_NB - 1) * _BLOCK
_NSLOTS = 12
_NTHREADS = 2


def _onehot_body(ids_ref, out_ref, scratch, sems):
    i = pl.program_id(0)
    slot = jax.lax.rem(i, _NSLOTS)

    @pl.when(i >= _NSLOTS)
    def _wait_prev():
        old = i - _NSLOTS
        pltpu.make_async_copy(
            scratch.at[slot],
            out_ref.at[pl.ds(old * _BLOCK, _BLOCK), :],
            sems.at[slot],
        ).wait()

    ids = ids_ref[...]
    iota = jax.lax.broadcasted_iota(jnp.int32, (_BLOCK, _D), 1)
    scratch[slot] = (ids == iota).astype(jnp.float32)

    @pl.when(i < _NB - 1)
    def _copy_full():
        for s in range(_NSLOTS):
            @pl.when(slot == s)
            def _start_one():
                pltpu.make_async_copy(
                    scratch.at[s],
                    out_ref.at[pl.ds(i * _BLOCK, _BLOCK), :],
                    sems.at[s],
                ).start(priority=s % _NTHREADS)

    @pl.when(i == _NB - 1)
    def _copy_last_and_drain():
        s_last = (_NB - 1) % _NSLOTS
        pltpu.make_async_copy(
            scratch.at[s_last, pl.ds(0, _LAST), :],
            out_ref.at[pl.ds((_NB - 1) * _BLOCK, _LAST), :],
            sems.at[s_last],
        ).start(priority=s_last % _NTHREADS)
        for step in range(max(0, _NB - _NSLOTS), _NB):
            s = step % _NSLOTS
            if step == _NB - 1:
                src = scratch.at[s, pl.ds(0, _LAST), :]
                dst = out_ref.at[pl.ds(step * _BLOCK, _LAST), :]
            else:
                src = scratch.at[s]
                dst = out_ref.at[pl.ds(step * _BLOCK, _BLOCK), :]
            pltpu.make_async_copy(src, dst, sems.at[s]).wait()


def kernel(input_ids):
    padded = jnp.pad(input_ids, ((0, 0), (1, 0)), constant_values=0)
    flat = padded.reshape(-1)
    flat = jnp.pad(flat, (0, _NB * _BLOCK - _ROWS), constant_values=-1)
    ids_col = flat.reshape(_NB * _BLOCK, 1)
    out = pl.pallas_call(
        _onehot_body,
        grid=(_NB,),
        in_specs=[pl.BlockSpec((_BLOCK, 1), lambda i: (i, 0))],
        out_specs=pl.BlockSpec(memory_space=pltpu.MemorySpace.HBM),
        out_shape=jax.ShapeDtypeStruct((_ROWS, _D), jnp.float32),
        scratch_shapes=[
            pltpu.VMEM((_NSLOTS, _BLOCK, _D), jnp.float32),
            pltpu.SemaphoreType.DMA((_NSLOTS,)),
        ],
    )(ids_col)
    return out.reshape(_B, _S, _D)


# speedup vs baseline: 1.1212x; 1.1212x over previous
import jax
import jax.numpy as jnp
from jax.experimental import pallas as pl

_B = 4
_S = 8193
_D = 2048
_BLOCK = 512
_NB = (_S + _BLOCK - 1) // _BLOCK


def _onehot_body(ids_ref, out_ref):
    ids = ids_ref[0]
    iota = jax.lax.broadcasted_iota(jnp.int32, (_BLOCK, _D), 1)
    out_ref[0] = (ids == iota).astype(jnp.float32)


def kernel(input_ids):
    padded = jnp.pad(input_ids, ((0, 0), (1, 0)), constant_values=0)
    ids3 = jnp.pad(padded, ((0, 0), (0, _NB * _BLOCK - _S)),
                   constant_values=-1).reshape(_B, _NB * _BLOCK, 1)
    return pl.pallas_call(
        _onehot_body,
        grid=(_B, _NB),
        in_specs=[pl.BlockSpec((1, _BLOCK, 1), lambda b, i: (b, i, 0))],
        out_specs=pl.BlockSpec((1, _BLOCK, _D), lambda b, i: (b, i, 0)),
        out_shape=jax.ShapeDtypeStruct((_B, _S, _D), jnp.float32),
    )(ids3)


# speedup vs baseline: 1.1258x; 1.0041x over previous
import jax
import jax.numpy as jnp
from jax.experimental import pallas as pl

_B = 4
_S = 8193
_D = 2048
_BLOCK = 512
_NB = (_S + _BLOCK - 1) // _BLOCK


def _onehot_body(ids_ref, out_ref):
    ids = ids_ref[0, :, 0:1]
    iota = jax.lax.broadcasted_iota(jnp.int32, (_BLOCK, _D), 1)
    out_ref[0] = (ids == iota).astype(jnp.float32)


def kernel(input_ids):
    padded = jnp.pad(input_ids, ((0, 0), (1, 0)), constant_values=0)
    ids2 = jnp.pad(padded, ((0, 0), (0, _NB * _BLOCK - _S)),
                   constant_values=-1)
    ids3 = jnp.broadcast_to(ids2[:, :, None], (_B, _NB * _BLOCK, 128))
    return pl.pallas_call(
        _onehot_body,
        grid=(_B, _NB),
        in_specs=[pl.BlockSpec((1, _BLOCK, 128), lambda b, i: (b, i, 0))],
        out_specs=pl.BlockSpec((1, _BLOCK, _D), lambda b, i: (b, i, 0)),
        out_shape=jax.ShapeDtypeStruct((_B, _S, _D), jnp.float32),
    )(ids3)


# speedup vs baseline: 5.4535x; 4.8440x over previous
import jax
import jax.numpy as jnp
from jax.experimental import pallas as pl

_B = 4
_S = 8193
_D = 2048
_BLOCK = 512
_NB = (_S + _BLOCK - 1) // _BLOCK
_NP = _NB * _BLOCK


def _onehot_body(a_ref, out_ref):
    a = a_ref[...]
    iota = jax.lax.broadcasted_iota(jnp.int32, (_BLOCK, 64, 128), 2)
    out_ref[...] = (a[:, :, None] == iota).astype(jnp.float32)


def kernel(input_ids):
    padded = jnp.pad(input_ids, ((0, 0), (1, 0)), constant_values=0)
    pt = padded.T
    a = jnp.tile(pt, (1, 16))
    a = a - (jnp.arange(64, dtype=jnp.int32) // 4 * 128)[None, :]
    a = jnp.pad(a, ((0, _NP - _S), (0, 0)), constant_values=-1)
    out3 = pl.pallas_call(
        _onehot_body,
        grid=(_NB,),
        in_specs=[pl.BlockSpec((_BLOCK, 64), lambda i: (i, 0))],
        out_specs=pl.BlockSpec((_BLOCK, 64, 128), lambda i: (i, 0, 0)),
        out_shape=jax.ShapeDtypeStruct((_S, 64, 128), jnp.float32),
    )(a)
    out = out3.reshape(_S, 16, _B, 128).transpose(2, 0, 1, 3)
    return out.reshape(_B, _S, _D)
